# baseline (speedup 1.0000x reference)
import functools

import jax
import jax.numpy as jnp
from jax import lax
from jax.experimental import pallas as pl
from jax.experimental.pallas import tpu as pltpu
from jax.experimental.pallas import tpu_sc as plsc


def _make_sc_gather(B0, S, V, D, n_workers, nc):
    L = 128
    SP = 56
    rows_per_w = B0 // n_workers
    CR = 8
    n_chunks = rows_per_w // CR
    mesh = plsc.VectorSubcoreMesh(core_axis_name="c", subcore_axis_name="s")

    @functools.partial(
        pl.kernel,
        mesh=mesh,
        compiler_params=pltpu.CompilerParams(use_tc_tiling_on_sc=False),
        out_type=jax.ShapeDtypeStruct((B0, S, D), jnp.float32),
        scratch_types=[
            pltpu.VMEM((rows_per_w, L), jnp.int32),
            pltpu.VMEM((CR * SP, D), jnp.float32),
            pltpu.VMEM((CR * SP, D), jnp.float32),
            pltpu.SemaphoreType.DMA,
            pltpu.SemaphoreType.DMA,
            pltpu.SemaphoreType.DMA,
            pltpu.SemaphoreType.DMA,
        ],
    )
    def k(ids_hbm, table_hbm, out_hbm, idx_all, rows0, rows1,
          semg0, semg1, sems0, sems1):
        wid = lax.axis_index("s") * nc + lax.axis_index("c")
        base = wid * rows_per_w
        pltpu.sync_copy(ids_hbm.at[pl.ds(base, rows_per_w)], idx_all)

        bufs = (rows0, rows1)
        gsems = (semg0, semg1)
        ssems = (sems0, sems1)

        def start_gathers(i):
            buf = bufs[i % 2]
            return [
                pltpu.async_copy(
                    table_hbm.at[idx_all.at[i * CR + j, pl.ds(0, SP)]],
                    buf.at[pl.ds(j * SP, SP)],
                    gsems[i % 2],
                )
                for j in range(CR)
            ]

        def start_stores(i):
            buf = bufs[i % 2]
            return [
                pltpu.async_copy(
                    buf.at[pl.ds(j * SP, S)],
                    out_hbm.at[base + i * CR + j],
                    ssems[i % 2],
                )
                for j in range(CR)
            ]

        gcps = [None] * n_chunks
        scps = [None] * n_chunks
        gcps[0] = start_gathers(0)
        for i in range(n_chunks):
            for cp in gcps[i]:
                cp.wait()
            if i >= 1:
                for cp in scps[i - 1]:
                    cp.wait()
            if i + 1 < n_chunks:
                gcps[i + 1] = start_gathers(i + 1)
            scps[i] = start_stores(i)
        for cp in scps[n_chunks - 1]:
            cp.wait()

    return k


def kernel(entity_ids, table):
    B0, S = entity_ids.shape
    V, D = table.shape
    info = plsc.get_sparse_core_info()
    n_workers = info.num_cores * info.num_subcores
    ids_pad = jnp.pad(entity_ids.astype(jnp.int32), ((0, 0), (0, 128 - S)))
    return _make_sc_gather(B0, S, V, D, n_workers, info.num_cores)(ids_pad, table)


# speedup vs baseline: 2.5854x; 2.5854x over previous
import functools

import jax
import jax.numpy as jnp
from jax import lax
from jax.experimental import pallas as pl
from jax.experimental.pallas import tpu as pltpu
from jax.experimental.pallas import tpu_sc as plsc


def _make_sc_gather(B0, S, V, D, n_workers, nc):
    L = 128
    B = B0 * S
    rows_per_w = B0 // n_workers
    CR = 8
    n_chunks = rows_per_w // CR
    mesh = plsc.VectorSubcoreMesh(core_axis_name="c", subcore_axis_name="s")

    @functools.partial(
        pl.kernel,
        mesh=mesh,
        compiler_params=pltpu.CompilerParams(use_tc_tiling_on_sc=False),
        out_type=jax.ShapeDtypeStruct((B, L), jnp.float32),
        scratch_types=[
            pltpu.VMEM((rows_per_w, S), jnp.int32),
            pltpu.VMEM((CR * S, L), jnp.float32),
            pltpu.VMEM((CR * S, L), jnp.float32),
            pltpu.SemaphoreType.DMA,
            pltpu.SemaphoreType.DMA,
            pltpu.SemaphoreType.DMA,
            pltpu.SemaphoreType.DMA,
        ],
    )
    def k(ids_hbm, table_hbm, mid_hbm, idx_all, rows0, rows1,
          semg0, semg1, sems0, sems1):
        wid = lax.axis_index("s") * nc + lax.axis_index("c")
        base = wid * rows_per_w
        pltpu.sync_copy(ids_hbm.at[pl.ds(base, rows_per_w)], idx_all)

        bufs = (rows0, rows1)
        gsems = (semg0, semg1)
        ssems = (sems0, sems1)

        def start_gathers(i):
            buf = bufs[i % 2]
            return [
                pltpu.async_copy(
                    table_hbm.at[idx_all.at[i * CR + j]],
                    buf.at[pl.ds(j * S, S)],
                    gsems[i % 2],
                )
                for j in range(CR)
            ]

        def start_store(i):
            return pltpu.async_copy(
                bufs[i % 2],
                mid_hbm.at[pl.ds((base + i * CR) * S, CR * S)],
                ssems[i % 2],
            )

        gcps = [None] * n_chunks
        scps = [None] * n_chunks
        gcps[0] = start_gathers(0)
        for i in range(n_chunks):
            for cp in gcps[i]:
                cp.wait()
            if i >= 1:
                scps[i - 1].wait()
            if i + 1 < n_chunks:
                gcps[i + 1] = start_gathers(i + 1)
            scps[i] = start_store(i)
        scps[n_chunks - 1].wait()

    return k


def _make_tc_finish(B0, S, D):
    L = 128
    RB = 128
    grid = B0 // RB

    def body(mid_ref, out_ref):
        y = mid_ref[...]
        out_ref[...] = y[:, :D].reshape(RB, S, D)

    return pl.pallas_call(
        body,
        grid=(grid,),
        in_specs=[pl.BlockSpec((RB * S, L), lambda i: (i, 0))],
        out_specs=pl.BlockSpec((RB, S, D), lambda i: (i, 0, 0)),
        out_shape=jax.ShapeDtypeStruct((B0, S, D), jnp.float32),
    )


def kernel(entity_ids, table):
    B0, S = entity_ids.shape
    V, D = table.shape
    info = plsc.get_sparse_core_info()
    n_workers = info.num_cores * info.num_subcores
    ids = entity_ids.astype(jnp.int32)
    tpad = jnp.pad(table, ((0, 0), (0, 128 - D)))
    mid = _make_sc_gather(B0, S, V, D, n_workers, info.num_cores)(ids, tpad)
    return _make_tc_finish(B0, S, D)(mid)


# speedup vs baseline: 3.2012x; 1.2382x over previous
import functools

import jax
import jax.numpy as jnp
from jax import lax
from jax.experimental import pallas as pl
from jax.experimental.pallas import tpu as pltpu
from jax.experimental.pallas import tpu_sc as plsc


def _make_sc_gather(B0, S, V, D, n_workers, nc):
    L = 128
    B = B0 * S
    rows_per_w = B0 // n_workers
    CR = 8
    n_chunks = rows_per_w // CR
    mesh = plsc.VectorSubcoreMesh(core_axis_name="c", subcore_axis_name="s")

    @functools.partial(
        pl.kernel,
        mesh=mesh,
        compiler_params=pltpu.CompilerParams(use_tc_tiling_on_sc=False),
        out_type=jax.ShapeDtypeStruct((B, L), jnp.float32),
        scratch_types=[
            pltpu.VMEM((rows_per_w, S), jnp.int32),
            pltpu.VMEM((CR * S, L), jnp.float32),
            pltpu.VMEM((CR * S, L), jnp.float32),
            pltpu.SemaphoreType.DMA,
            pltpu.SemaphoreType.DMA,
            pltpu.SemaphoreType.DMA,
            pltpu.SemaphoreType.DMA,
        ],
    )
    def k(ids_hbm, table_hbm, mid_hbm, idx_all, rows0, rows1,
          semg0, semg1, sems0, sems1):
        wid = lax.axis_index("s") * nc + lax.axis_index("c")
        base = wid * rows_per_w
        pltpu.sync_copy(ids_hbm.at[pl.ds(base, rows_per_w)], idx_all)

        bufs = (rows0, rows1)
        gsems = (semg0, semg1)
        ssems = (sems0, sems1)

        def start_gathers(i):
            buf = bufs[i % 2]
            return [
                pltpu.async_copy(
                    table_hbm.at[idx_all.at[i * CR + j]],
                    buf.at[pl.ds(j * S, S)],
                    gsems[i % 2],
                )
                for j in range(CR)
            ]

        def start_store(i):
            return pltpu.async_copy(
                bufs[i % 2],
                mid_hbm.at[pl.ds((base + i * CR) * S, CR * S)],
                ssems[i % 2],
            )

        gcps = [None] * n_chunks
        scps = [None] * n_chunks
        gcps[0] = start_gathers(0)
        for i in range(n_chunks):
            for cp in gcps[i]:
                cp.wait()
            if i >= 1:
                scps[i - 1].wait()
            if i + 1 < n_chunks:
                gcps[i + 1] = start_gathers(i + 1)
            scps[i] = start_store(i)
        scps[n_chunks - 1].wait()

    return k


def _make_tc_finish(B0, S, D):
    L = 128
    RB = 256
    grid = B0 // RB

    def body(mid_ref, out_ref):
        y = mid_ref[...]
        z = y[:, :D].reshape(RB, S, D)
        out_ref[...] = jnp.transpose(z, (1, 2, 0))

    return pl.pallas_call(
        body,
        grid=(grid,),
        in_specs=[pl.BlockSpec((RB * S, L), lambda i: (i, 0))],
        out_specs=pl.BlockSpec((S, D, RB), lambda i: (0, 0, i)),
        out_shape=jax.ShapeDtypeStruct((S, D, B0), jnp.float32),
    )


def kernel(entity_ids, table):
    B0, S = entity_ids.shape
    V, D = table.shape
    info = plsc.get_sparse_core_info()
    n_workers = info.num_cores * info.num_subcores
    ids = entity_ids.astype(jnp.int32)
    tpad = jnp.pad(table, ((0, 0), (0, 128 - D)))
    mid = _make_sc_gather(B0, S, V, D, n_workers, info.num_cores)(ids, tpad)
    out_t = _make_tc_finish(B0, S, D)(mid)
    return out_t.transpose(2, 0, 1)
